# Initial kernel scaffold; baseline (speedup 1.0000x reference)
import jax
import jax.numpy as jnp
from jax.experimental import pallas as pl


def kernel(x, embed, W, b):
    raise NotImplementedError("write your pallas kernel here")


# speedup vs baseline: 413.1956x; 413.1956x over previous
import functools

import jax
import jax.numpy as jnp
from jax import lax
from jax.experimental import pallas as pl
from jax.experimental.pallas import tpu as pltpu
from jax.experimental.pallas import tpu_sc as plsc

_NC = 2
_NS = 16
_NW = _NC * _NS
_L = 16

_B = 16384
_D = 128
_BPW = _B // _NW
_CHUNK = 128
_NCHUNK = _BPW // _CHUNK
_NGROUP = _BPW // _L


def _sc_body(embed_hbm, idx_hbm, w_hbm, b_hbm, out_hbm,
             idx_v, rows_v, w_v, b_v, out_v, sem):
    wid = lax.axis_index("s") * _NC + lax.axis_index("c")
    base = wid * _BPW

    pltpu.sync_copy(idx_hbm.at[wid], idx_v)
    pltpu.sync_copy(w_hbm, w_v)
    pltpu.sync_copy(b_hbm, b_v)

    for k in range(_NCHUNK):
        pltpu.make_async_copy(
            embed_hbm.at[idx_v.at[k]],
            rows_v.at[pl.ds(k * _CHUNK, _CHUNK)],
            sem,
        ).start()
    for k in range(_NCHUNK):
        pltpu.make_async_copy(
            embed_hbm.at[idx_v.at[k]],
            rows_v.at[pl.ds(k * _CHUNK, _CHUNK)],
            sem,
        ).wait()

    wc = [w_v[pl.ds(16 * c, 16)] for c in range(_D // _L)]

    lanes = lax.iota(jnp.int32, 16)
    perms = [lanes ^ (1 << k) for k in range(4)]
    _dnums = lax.GatherDimensionNumbers(
        offset_dims=(), collapsed_slice_dims=(0,), start_index_map=(0,))

    def _permute(v, pm):
        return lax.gather(
            v, pm[:, None], dimension_numbers=_dnums, slice_sizes=(1,),
            mode=lax.GatherScatterMode.PROMISE_IN_BOUNDS)

    bias = b_v[:]

    def group_body(g, carry):
        acc = bias
        for k in range(_L):
            i = g * _L + k
            p = [rows_v[i, pl.ds(16 * c, 16)] * wc[c] for c in range(_D // _L)]
            s = (((p[0] + p[1]) + (p[2] + p[3]))
                 + ((p[4] + p[5]) + (p[6] + p[7])))
            for pm in perms:
                s = s + _permute(s, pm)
            acc = acc + jnp.where(lanes == k, s, 0.0)
        h = jnp.maximum(acc, 0.0)
        r = 1.0 / (1.0 + jnp.exp(-h))
        out_v[pl.ds(g * _L, 16)] = r
        return carry

    lax.fori_loop(0, _NGROUP, group_body, 0)

    pltpu.sync_copy(out_v, out_hbm.at[pl.ds(base, _BPW)])


@jax.jit
def _classify(embed, idx3, w, b16):
    mesh = plsc.VectorSubcoreMesh(core_axis_name="c", subcore_axis_name="s")
    f = functools.partial(
        pl.kernel,
        mesh=mesh,
        out_type=jax.ShapeDtypeStruct((_B,), jnp.float32),
        scratch_types=[
            pltpu.VMEM((_NCHUNK, _CHUNK), jnp.int32),
            pltpu.VMEM((_BPW, _D), jnp.float32),
            pltpu.VMEM((_D,), jnp.float32),
            pltpu.VMEM((_L,), jnp.float32),
            pltpu.VMEM((_BPW,), jnp.float32),
            pltpu.SemaphoreType.DMA,
        ],
    )(_sc_body)
    return f(embed, idx3, w, b16)


def kernel(x, embed, W, b):
    idx = x[:, 0].astype(jnp.int32)
    idx3 = idx.reshape(_NW, _NCHUNK, _CHUNK)
    w = W.reshape(_D).astype(jnp.float32)
    b16 = jnp.broadcast_to(b.astype(jnp.float32).reshape(()), (_L,))
    out = _classify(embed.astype(jnp.float32), idx3, w, b16)
    return out.reshape(_B, 1)
